# Initial kernel scaffold; baseline (speedup 1.0000x reference)
import jax
import jax.numpy as jnp
from jax.experimental import pallas as pl


def kernel(item_his, time_his, cate_his, shop_his, node_his, product_his, brand_his, item_id, time_id, cate_id, shop_id, node_id, product_id, brand_id, mask, item_table, cate_table, shop_table, node_table, product_table, brand_table, time_table):
    raise NotImplementedError("write your pallas kernel here")


# speedup vs baseline: 6.3454x; 6.3454x over previous
import jax
import jax.numpy as jnp
from jax import lax
from jax.experimental import pallas as pl
from jax.experimental.pallas import tpu as pltpu, tpu_sc as plsc

NC, NS = 2, 16
NW = NC * NS
B, L, D, TD = 1024, 200, 32, 8
BW = B // NW
CH0, CH1 = 128, 72
OW = 12 * D + 2 * TD
HIS0 = 6 * D
T0 = 12 * D

_BCAST_DNUMS = lax.GatherDimensionNumbers(
    offset_dims=(), collapsed_slice_dims=(0,), start_index_map=(0,))


def _vgather(vec, idx):
  return lax.gather(vec, idx[:, None], _BCAST_DNUMS, slice_sizes=(1,),
                    mode=lax.GatherScatterMode.PROMISE_IN_BOUNDS)


def _body(item_his, time_his, cate_his, shop_his, node_his, product_his,
          brand_his, item_id, time_id, cate_id, shop_id, node_id, product_id,
          brand_id, mask, item_table, cate_table, shop_table, node_table,
          product_table, brand_table, time_table, out,
          idx7, mask_v, rows, idb, idrows, out_v, sem):
  wid = lax.axis_index("s") * NC + lax.axis_index("c")
  base = wid * BW
  his_arrs = [item_his, cate_his, shop_his, node_his, product_his, brand_his,
              time_his]
  tables = [item_table, cate_table, shop_table, node_table, product_table,
            brand_table, time_table]
  id_arrs = [item_id, cate_id, shop_id, node_id, product_id, brand_id,
             time_id]
  iota = lax.iota(jnp.int32, 16)
  zero = jnp.zeros((16,), jnp.float32)

  dd = [pltpu.async_copy(a.at[pl.ds(base, BW)], idb.at[pl.ds(BW * f, BW)],
                         sem)
        for f, a in enumerate(id_arrs)]
  for dsc in dd:
    dsc.wait()
  gg = [pltpu.async_copy(t.at[idb.at[pl.ds(BW * f, BW)]], idrows.at[f], sem)
        for f, t in enumerate(tables)]
  for dsc in gg:
    dsc.wait()

  def bstep(b, carry):
    bg = base + b
    ob = OW * b
    descs = [pltpu.async_copy(a.at[bg], idx7.at[pl.ds(L * f, L)], sem)
             for f, a in enumerate(his_arrs)]
    descs.append(pltpu.async_copy(mask.at[bg], mask_v.at[pl.ds(0, L)], sem))
    for dsc in descs:
      dsc.wait()

    g = []
    for f, tab in enumerate(tables):
      g.append(pltpu.async_copy(tab.at[idx7.at[pl.ds(L * f, CH0)]],
                                rows.at[f, pl.ds(0, CH0), :], sem))
      g.append(pltpu.async_copy(tab.at[idx7.at[pl.ds(L * f + CH0, CH1)]],
                                rows.at[f, pl.ds(CH0, CH1), :], sem))
    for dsc in g:
      dsc.wait()

    for f in range(6):
      out_v[pl.ds(ob + D * f, 16)] = idrows[f, b, pl.ds(0, 16)]
      out_v[pl.ds(ob + D * f + 16, 16)] = idrows[f, b, pl.ds(16, 16)]

    def lstep(l, accs):
      mvec = mask_v[pl.ds((l // 16) * 16, 16)]
      m = _vgather(mvec, jnp.full((16,), l % 16, jnp.int32))
      new = []
      for f in range(6):
        lo = accs[2 * f] + m * rows[f, l, pl.ds(0, 16)]
        hi = accs[2 * f + 1] + m * rows[f, l, pl.ds(16, 16)]
        new += [lo, hi]
      new.append(accs[12] + m * rows[6, l, pl.ds(0, 16)])
      return tuple(new)

    accs = lax.fori_loop(0, L, lstep, (zero,) * 13)
    for f in range(6):
      out_v[pl.ds(ob + HIS0 + D * f, 16)] = accs[2 * f]
      out_v[pl.ds(ob + HIS0 + D * f + 16, 16)] = accs[2 * f + 1]

    tsum = _vgather(accs[12], jnp.maximum(iota - 8, 0))
    tid16 = idrows[6, b, pl.ds(0, 16)]
    out_v[pl.ds(ob + T0, 16)] = jnp.where(iota < 8, tid16, tsum)
    return carry

  lax.fori_loop(0, BW, bstep, 0)
  pltpu.sync_copy(out_v, out.at[pl.ds(base * OW, BW * OW)])


def kernel(item_his, time_his, cate_his, shop_his, node_his, product_his,
           brand_his, item_id, time_id, cate_id, shop_id, node_id,
           product_id, brand_id, mask, item_table, cate_table, shop_table,
           node_table, product_table, brand_table, time_table):
  mesh = plsc.VectorSubcoreMesh(core_axis_name="c", subcore_axis_name="s",
                                num_cores=NC, num_subcores=NS)
  time_table_p = jnp.pad(time_table, ((0, 0), (0, D - TD)))
  run = pl.kernel(
      _body,
      out_type=jax.ShapeDtypeStruct((B * OW,), jnp.float32),
      mesh=mesh,
      compiler_params=pltpu.CompilerParams(use_tc_tiling_on_sc=False),
      scratch_types=[
          pltpu.VMEM((7 * L,), jnp.int32),
          pltpu.VMEM((208,), jnp.float32),
          pltpu.VMEM((7, L, D), jnp.float32),
          pltpu.VMEM((7 * BW,), jnp.int32),
          pltpu.VMEM((7, BW, D), jnp.float32),
          pltpu.VMEM((BW * OW,), jnp.float32),
          pltpu.SemaphoreType.DMA,
      ],
  )
  flat = run(item_his, time_his, cate_his, shop_his, node_his, product_his,
             brand_his, item_id, time_id, cate_id, shop_id, node_id,
             product_id, brand_id, mask, item_table, cate_table, shop_table,
             node_table, product_table, brand_table, time_table_p)
  return flat.reshape(B, OW)


# speedup vs baseline: 6.6164x; 1.0427x over previous
import jax
import jax.numpy as jnp
from jax import lax
from jax.experimental import pallas as pl
from jax.experimental.pallas import tpu as pltpu, tpu_sc as plsc

NC, NS = 2, 16
NW = NC * NS
B, L, D, TD = 1024, 200, 32, 8
BW = B // NW
CH0, CH1 = 128, 72
OW = 12 * D + 2 * TD
HIS0 = 6 * D
T0 = 12 * D
NCH = L // 16
TAIL = L - 16 * NCH

_BCAST_DNUMS = lax.GatherDimensionNumbers(
    offset_dims=(), collapsed_slice_dims=(0,), start_index_map=(0,))


def _vgather(vec, idx):
  return lax.gather(vec, idx[:, None], _BCAST_DNUMS, slice_sizes=(1,),
                    mode=lax.GatherScatterMode.PROMISE_IN_BOUNDS)


def _body(item_his, time_his, cate_his, shop_his, node_his, product_his,
          brand_his, item_id, time_id, cate_id, shop_id, node_id, product_id,
          brand_id, mask, item_table, cate_table, shop_table, node_table,
          product_table, brand_table, time_table, out,
          idx_s, mask_s, rows_s, idb, idrows, out_v,
          sem_st, sem_g, sem_id):
  wid = lax.axis_index("s") * NC + lax.axis_index("c")
  base = wid * BW
  his_arrs = [item_his, cate_his, shop_his, node_his, product_his, brand_his,
              time_his]
  tables = [item_table, cate_table, shop_table, node_table, product_table,
            brand_table, time_table]
  id_arrs = [item_id, cate_id, shop_id, node_id, product_id, brand_id,
             time_id]
  iota = lax.iota(jnp.int32, 16)
  zero = jnp.zeros((16,), jnp.float32)
  ksplat = [jnp.full((16,), k, jnp.int32) for k in range(16)]

  dd = [pltpu.async_copy(a.at[pl.ds(base, BW)], idb.at[pl.ds(BW * f, BW)],
                         sem_id)
        for f, a in enumerate(id_arrs)]
  for dsc in dd:
    dsc.wait()
  gg = [pltpu.async_copy(t.at[idb.at[pl.ds(BW * f, BW)]], idrows.at[f],
                         sem_id)
        for f, t in enumerate(tables)]
  for dsc in gg:
    dsc.wait()

  def stage_descs(b):
    bc = jnp.minimum(b, BW - 1)
    bg = base + bc
    sb = bc & 3
    d = [pltpu.make_async_copy(a.at[bg], idx_s.at[sb, pl.ds(L * f, L)],
                               sem_st)
         for f, a in enumerate(his_arrs)]
    d.append(pltpu.make_async_copy(mask.at[bg], mask_s.at[sb, pl.ds(0, L)],
                                   sem_st))
    return d

  def gather_descs(b, s):
    sb = jnp.minimum(b, BW - 1) & 3
    d = []
    for f, tab in enumerate(tables):
      d.append(pltpu.make_async_copy(tab.at[idx_s.at[sb, pl.ds(L * f, CH0)]],
                                     rows_s.at[s, f, pl.ds(0, CH0), :],
                                     sem_g[s]))
      d.append(pltpu.make_async_copy(
          tab.at[idx_s.at[sb, pl.ds(L * f + CH0, CH1)]],
          rows_s.at[s, f, pl.ds(CH0, CH1), :], sem_g[s]))
    return d

  def fire_stage(b):
    for dsc in stage_descs(b):
      dsc.start()

  def wait_stage(b):
    for dsc in stage_descs(b):
      dsc.wait()

  def fire_gather(b, s):
    for dsc in gather_descs(b, s):
      dsc.start()

  def wait_gather(b, s):
    for dsc in gather_descs(b, s):
      dsc.wait()

  def compute(b, s):
    ob = OW * b
    for f in range(6):
      out_v[pl.ds(ob + D * f, 16)] = idrows[f, b, pl.ds(0, 16)]
      out_v[pl.ds(ob + D * f + 16, 16)] = idrows[f, b, pl.ds(16, 16)]

    sb = b & 3
    def chunk(c, accs):
      base_l = 16 * c
      mvec = mask_s[sb, pl.ds(base_l, 16)]
      accs = list(accs)
      for k in range(16):
        m = _vgather(mvec, ksplat[k])
        l = base_l + k
        for f in range(6):
          accs[2 * f] = accs[2 * f] + m * rows_s[s, f, l, pl.ds(0, 16)]
          accs[2 * f + 1] = (accs[2 * f + 1] +
                             m * rows_s[s, f, l, pl.ds(16, 16)])
        accs[12] = accs[12] + m * rows_s[s, 6, l, pl.ds(0, 16)]
      return tuple(accs)

    accs = list(lax.fori_loop(0, NCH, chunk, (zero,) * 13))
    mvec = mask_s[sb, pl.ds(16 * NCH, 16)]
    for k in range(TAIL):
      m = _vgather(mvec, ksplat[k])
      l = 16 * NCH + k
      for f in range(6):
        accs[2 * f] = accs[2 * f] + m * rows_s[s, f, l, pl.ds(0, 16)]
        accs[2 * f + 1] = (accs[2 * f + 1] +
                           m * rows_s[s, f, l, pl.ds(16, 16)])
      accs[12] = accs[12] + m * rows_s[s, 6, l, pl.ds(0, 16)]

    for f in range(6):
      out_v[pl.ds(ob + HIS0 + D * f, 16)] = accs[2 * f]
      out_v[pl.ds(ob + HIS0 + D * f + 16, 16)] = accs[2 * f + 1]

    tsum = _vgather(accs[12], jnp.maximum(iota - 8, 0))
    tid16 = idrows[6, b, pl.ds(0, 16)]
    out_v[pl.ds(ob + T0, 16)] = jnp.where(iota < 8, tid16, tsum)

  fire_stage(0)
  wait_stage(0)
  fire_gather(0, 0)
  fire_stage(1)

  def pipe(g, carry):
    b0 = 2 * g
    wait_stage(b0 + 1)
    fire_gather(b0 + 1, 1)
    wait_gather(b0, 0)
    fire_stage(b0 + 2)
    compute(b0, 0)
    wait_stage(b0 + 2)
    fire_gather(b0 + 2, 0)
    wait_gather(b0 + 1, 1)
    fire_stage(b0 + 3)
    compute(b0 + 1, 1)
    return carry

  lax.fori_loop(0, BW // 2, pipe, 0)
  wait_gather(BW, 0)
  wait_stage(BW + 1)

  pltpu.sync_copy(out_v, out.at[pl.ds(base * OW, BW * OW)])


def kernel(item_his, time_his, cate_his, shop_his, node_his, product_his,
           brand_his, item_id, time_id, cate_id, shop_id, node_id,
           product_id, brand_id, mask, item_table, cate_table, shop_table,
           node_table, product_table, brand_table, time_table):
  mesh = plsc.VectorSubcoreMesh(core_axis_name="c", subcore_axis_name="s",
                                num_cores=NC, num_subcores=NS)
  time_table_p = jnp.pad(time_table, ((0, 0), (0, D - TD)))
  run = pl.kernel(
      _body,
      out_type=jax.ShapeDtypeStruct((B * OW,), jnp.float32),
      mesh=mesh,
      compiler_params=pltpu.CompilerParams(use_tc_tiling_on_sc=False),
      scratch_types=[
          pltpu.VMEM((4, 7 * L), jnp.int32),
          pltpu.VMEM((4, 208), jnp.float32),
          pltpu.VMEM((2, 7, L, D), jnp.float32),
          pltpu.VMEM((7 * BW,), jnp.int32),
          pltpu.VMEM((7, BW, D), jnp.float32),
          pltpu.VMEM((BW * OW,), jnp.float32),
          pltpu.SemaphoreType.DMA,
          [pltpu.SemaphoreType.DMA] * 2,
          pltpu.SemaphoreType.DMA,
      ],
  )
  flat = run(item_his, time_his, cate_his, shop_his, node_his, product_his,
             brand_his, item_id, time_id, cate_id, shop_id, node_id,
             product_id, brand_id, mask, item_table, cate_table, shop_table,
             node_table, product_table, brand_table, time_table_p)
  return flat.reshape(B, OW)
